# baseline (speedup 1.0000x reference)
import functools

import jax
import jax.numpy as jnp
from jax import lax
from jax.experimental import pallas as pl
from jax.experimental.pallas import tpu as pltpu
from jax.experimental.pallas import tpu_sc as plsc

F = 26
V = 100000
D = 32
B = 16384
NC = 2
NS = 16
NW = NC * NS
RPW = B // NW
CH = 128
NCH = RPW // CH


def _fm_body(xt_hbm, emb_hbm, lin_hbm, bias_hbm, out_hbm,
             idx_v, rows_v, lin_v, out_v, bias_v, sem_e, sem_l):
    wid = lax.axis_index("s") * NC + lax.axis_index("c")
    base_row = wid * RPW

    pltpu.sync_copy(bias_hbm, bias_v.at[pl.ds(0, 1)])
    bias_s = bias_v[...][0]

    lane = lax.iota(jnp.int32, 16)
    zero16f = jnp.zeros((16,), jnp.float32)

    def chunk_body(c, carry):
        row0 = base_row + c * CH
        pltpu.sync_copy(xt_hbm.at[:, pl.ds(row0, CH)], idx_v)
        cps = []
        for f in range(F):
            cps.append(pltpu.async_copy(emb_hbm.at[f].at[idx_v.at[f]],
                                        rows_v.at[pl.ds(f * CH, CH)], sem_e))
            cps.append(pltpu.async_copy(lin_hbm.at[f].at[idx_v.at[f]],
                                        lin_v.at[pl.ds(f * CH, CH)], sem_l))
        for cp in cps:
            cp.wait()

        def group_body(g, carry2):
            g16 = g * 16
            lin_acc = zero16f
            for f in range(F):
                lin_acc = lin_acc + lin_v[pl.ds(f * CH + g16, 16)]
            zacc = zero16f
            for rr in range(16):
                r = g16 + rr
                sa = zero16f
                sb = zero16f
                qa = zero16f
                qb = zero16f
                for f in range(F):
                    a = rows_v[f * CH + r, pl.ds(0, 16)]
                    b = rows_v[f * CH + r, pl.ds(16, 16)]
                    sa = sa + a
                    sb = sb + b
                    qa = qa + a * a
                    qb = qb + b * b
                p = sa * sa + sb * sb - qa - qb
                zacc = jnp.where(lane == rr, jnp.sum(p), zacc)
            z = zacc * 0.5 + lin_acc + bias_s
            out_v[pl.ds(g16, 16)] = 1.0 / (1.0 + jnp.exp(-z))
            return carry2

        lax.fori_loop(0, CH // 16, group_body, 0)
        pltpu.sync_copy(out_v, out_hbm.at[pl.ds(row0, CH)])
        return carry

    lax.fori_loop(0, NCH, chunk_body, 0)


@jax.jit
def _fm_call(xt, emb_tables, lin2, bias):
    mesh = plsc.VectorSubcoreMesh(core_axis_name="c", subcore_axis_name="s")
    kern = pl.kernel(
        _fm_body,
        out_type=jax.ShapeDtypeStruct((B,), jnp.float32),
        mesh=mesh,
        scratch_types=[
            pltpu.VMEM((F, CH), jnp.int32),
            pltpu.VMEM((F * CH, D), jnp.float32),
            pltpu.VMEM((F * CH,), jnp.float32),
            pltpu.VMEM((CH,), jnp.float32),
            pltpu.VMEM((16,), jnp.float32),
            pltpu.SemaphoreType.DMA,
            pltpu.SemaphoreType.DMA,
        ],
        compiler_params=pltpu.CompilerParams(
            needs_layout_passes=False, use_tc_tiling_on_sc=False),
    )
    return kern(xt, emb_tables, lin2, bias)


def kernel(x, emb_tables, lin_tables, bias):
    out = _fm_call(x.T, emb_tables, lin_tables.reshape(F, V), bias)
    return out.reshape(B, 1)
